# baseline (speedup 1.0000x reference)
import functools

import jax
import jax.numpy as jnp
from jax import lax
from jax.experimental import pallas as pl
from jax.experimental.pallas import tpu as pltpu
from jax.experimental.pallas import tpu_sc as plsc

BATCH = 4096
SEQ = 200
EMBED = 64
LANES = 16

NC, NS = 2, 16
NW = NC * NS
TOTAL = BATCH * SEQ
PER_W = TOTAL // NW
SEQ_PER_BLK = 4
RPB = SEQ_PER_BLK * SEQ
NBLK = PER_W // RPB
GSZ = 100
NG = RPB // GSZ


def _pos_encoding(seq_len, d, n=10000.0):
    k = jnp.arange(seq_len, dtype=jnp.float32)[:, None]
    i = jnp.arange(d // 2, dtype=jnp.float32)[None, :]
    ang = k / jnp.power(n, 2.0 * i / d)
    p = jnp.zeros((seq_len, d), dtype=jnp.float32)
    p = p.at[:, 0::2].set(jnp.sin(ang))
    p = p.at[:, 1::2].set(jnp.cos(ang))
    return p


def _make_sc_kernel():
    mesh = plsc.VectorSubcoreMesh(core_axis_name="c", subcore_axis_name="s")

    @functools.partial(
        pl.kernel,
        mesh=mesh,
        compiler_params=pltpu.CompilerParams(use_tc_tiling_on_sc=False),
        out_type=jax.ShapeDtypeStruct((NW, NBLK, RPB, EMBED), jnp.float32),
        scratch_types=[
            pltpu.VMEM((SEQ, EMBED), jnp.float32),
            pltpu.VMEM((NG, GSZ), jnp.int32),
            pltpu.VMEM((NG, GSZ), jnp.int32),
            pltpu.VMEM((RPB, EMBED), jnp.float32),
            pltpu.VMEM((RPB, EMBED), jnp.float32),
            pltpu.SemaphoreType.DMA,
            pltpu.SemaphoreType.DMA,
            pltpu.SemaphoreType.DMA,
            pltpu.SemaphoreType.DMA,
        ],
    )
    def emb_kernel(table_hbm, x_hbm, p_hbm, out_hbm,
                   p_v, idx0, idx1, rows0, rows1,
                   semg0, semg1, semw0, semw1):
        wid = lax.axis_index("s") * NC + lax.axis_index("c")
        idx = (idx0, idx1)
        rows = (rows0, rows1)
        semg = (semg0, semg1)
        semw = (semw0, semw1)

        pltpu.sync_copy(p_hbm, p_v)

        def fire_gathers(buf):
            return [
                pltpu.async_copy(
                    table_hbm.at[idx[buf].at[j]],
                    rows[buf].at[pl.ds(j * GSZ, GSZ)],
                    semg[buf],
                )
                for j in range(NG)
            ]

        pltpu.sync_copy(x_hbm.at[wid, 0], idx[0])
        gathers = [fire_gathers(0), None]
        wb = [None, None]

        for b in range(NBLK):
            cur = b & 1
            nxt = cur ^ 1
            if b + 1 < NBLK:
                pltpu.sync_copy(x_hbm.at[wid, b + 1], idx[nxt])
                if wb[nxt] is not None:
                    wb[nxt].wait()
                gathers[nxt] = fire_gathers(nxt)
            for h in gathers[cur]:
                h.wait()

            rv = rows[cur]

            def pbody(p, carry, rv=rv):
                for d in range(EMBED // LANES):
                    pe = p_v[p, pl.ds(d * LANES, LANES)]
                    for s in range(SEQ_PER_BLK):
                        r = s * SEQ + p
                        rv[r, pl.ds(d * LANES, LANES)] = (
                            rv[r, pl.ds(d * LANES, LANES)] + pe)
                return carry

            lax.fori_loop(0, SEQ, pbody, 0)
            wb[cur] = pltpu.async_copy(rv, out_hbm.at[wid, b], semw[cur])

        for h in wb:
            if h is not None:
                h.wait()

    return emb_kernel


def kernel(table, x):
    p = _pos_encoding(SEQ, EMBED)
    x_r = x.reshape(NW, NBLK, NG, GSZ)
    out = _make_sc_kernel()(table, x_r, p)
    return out.reshape(BATCH, SEQ, EMBED)
